# baseline (speedup 1.0000x reference)
import functools

import jax
import jax.numpy as jnp
from jax import lax
from jax.experimental import pallas as pl
from jax.experimental.pallas import tpu as pltpu
from jax.experimental.pallas import tpu_sc as plsc

N = 10000
G = 8
NPG = N // G
E = 320000
D = 128
C = 64
K = 16
HID = 32

NC = 2
NS = 16
NW = NC * NS
EPW = E // NW
CH = 80
NCH = EPW // CH
NBUF = 5
PRE = 2
SRW = 624
TAIL = N - NS * SRW


def _mlp_body(x_ref, w1_ref, b1_ref, w2_ref, b2_ref, assign_ref, coarse_ref):
    w1 = w1_ref[...]
    b1 = b1_ref[0]
    w2 = w2_ref[...]
    b2 = b2_ref[0]
    for g in range(G):
        x = x_ref[pl.ds(g * NPG, NPG), :]
        h = jnp.dot(x, w1, preferred_element_type=jnp.float32)
        h = jnp.maximum(h + b1, 0.0)
        logits = jnp.dot(h, w2, preferred_element_type=jnp.float32) + b2
        m = jnp.max(logits, axis=-1, keepdims=True)
        e = jnp.exp(logits - m)
        a = e / jnp.sum(e, axis=-1, keepdims=True)
        assign_ref[pl.ds(g * NPG, NPG), :] = a
        coarse_ref[pl.ds(g * C, C), :] = lax.dot_general(
            a, x, (((0,), (0,)), ((), ())), preferred_element_type=jnp.float32)


def _stage1(nodes, w1, b1, w2, b2):
    return pl.pallas_call(
        _mlp_body,
        out_shape=[
            jax.ShapeDtypeStruct((N, C), jnp.float32),
            jax.ShapeDtypeStruct((G * C, D), jnp.float32),
        ],
    )(nodes, w1, b1, w2, b2)


def _edge_body(assign_hbm, senders_hbm, receivers_hbm, zeros_hbm, t_hbm,
               sidx_v, ridx_v, sbuf0, sbuf1, sbuf2, sbuf3, sbuf4,
               rows_v, t_sh, gsem, ssem):
    sbufs = (sbuf0, sbuf1, sbuf2, sbuf3, sbuf4)
    cid = lax.axis_index("c")
    sid = lax.axis_index("s")
    wid = sid * NC + cid
    stripe = pl.multiple_of(sid * SRW, 8)
    pltpu.sync_copy(zeros_hbm, t_sh.at[pl.ds(stripe, SRW)])

    @pl.when(sid == NS - 1)
    def _zero_tail():
        pltpu.sync_copy(zeros_hbm.at[pl.ds(0, TAIL)],
                        t_sh.at[pl.ds(NS * SRW, TAIL)])

    pltpu.sync_copy(senders_hbm.at[pl.ds(wid * EPW, EPW)], sidx_v)
    pltpu.sync_copy(receivers_hbm.at[pl.ds(wid * EPW, EPW)], ridx_v)
    plsc.subcore_barrier()

    for b in range(PRE):
        pltpu.make_async_copy(assign_hbm.at[ridx_v.at[pl.ds(b * CH, CH)]],
                              rows_v.at[b], gsem.at[b]).start()

    def outer(g, carry):
        for b in range(NBUF):
            c = g * NBUF + b
            pltpu.make_async_copy(assign_hbm.at[ridx_v.at[pl.ds(b * CH, CH)]],
                                  rows_v.at[b], gsem.at[b]).wait()
            for j in range(CH // 16):
                sbufs[b][pl.ds(j * 16, 16)] = sidx_v[pl.ds(c * CH + j * 16, 16)]
            pltpu.async_copy(rows_v.at[b], t_sh.at[sbufs[b]], ssem.at[b],
                             add=True)
            cn = c + PRE
            bp = (b + PRE) % NBUF

            @pl.when(cn < NCH)
            def _prefetch():
                @pl.when(cn >= NBUF)
                def _reclaim():
                    pltpu.make_async_copy(rows_v.at[bp], t_sh.at[sbufs[bp]],
                                          ssem.at[bp]).wait()

                pltpu.make_async_copy(
                    assign_hbm.at[ridx_v.at[pl.ds(cn * CH, CH)]],
                    rows_v.at[bp], gsem.at[bp]).start()
        return carry

    lax.fori_loop(0, NCH // NBUF, outer, 0)
    for b in range(NBUF):
        pltpu.make_async_copy(rows_v.at[b], t_sh.at[sbufs[b]],
                              ssem.at[b]).wait()
    plsc.subcore_barrier()
    pltpu.sync_copy(t_sh.at[pl.ds(stripe, SRW)],
                    t_hbm.at[cid, pl.ds(stripe, SRW)])

    @pl.when(sid == NS - 1)
    def _write_tail():
        pltpu.sync_copy(t_sh.at[pl.ds(NS * SRW, TAIL)],
                        t_hbm.at[cid, pl.ds(NS * SRW, TAIL)])


def _stage2(assignments, senders, receivers, zeros):
    run = pl.kernel(
        _edge_body,
        out_type=jax.ShapeDtypeStruct((NC, N, C), jnp.float32),
        mesh=plsc.VectorSubcoreMesh(core_axis_name="c", subcore_axis_name="s"),
        scratch_types=[
            pltpu.VMEM((EPW,), jnp.int32),
            pltpu.VMEM((EPW,), jnp.int32),
            pltpu.VMEM((CH,), jnp.int32),
            pltpu.VMEM((CH,), jnp.int32),
            pltpu.VMEM((CH,), jnp.int32),
            pltpu.VMEM((CH,), jnp.int32),
            pltpu.VMEM((CH,), jnp.int32),
            pltpu.VMEM((NBUF, CH, C), jnp.float32),
            pltpu.VMEM_SHARED((N, C), jnp.float32),
            pltpu.SemaphoreType.DMA((NBUF,)),
            pltpu.SemaphoreType.DMA((NBUF,)),
        ],
        compiler_params=pltpu.CompilerParams(use_tc_tiling_on_sc=False),
    )
    return run(assignments, senders, receivers, zeros)


def _adj_body(a_ref, t_ref, vals_ref, idx_ref):
    t = t_ref[0] + t_ref[1]
    a = a_ref[...]
    adjs = []
    for g in range(G):
        ag = a[g * NPG:(g + 1) * NPG]
        tg = t[g * NPG:(g + 1) * NPG]
        adjs.append(lax.dot_general(
            ag, tg, (((0,), (0,)), ((), ())),
            preferred_element_type=jnp.float32))
    work = jnp.concatenate(adjs, axis=0)
    jj = lax.broadcasted_iota(jnp.int32, (G * C, C), 1)
    vals_cols, idx_cols = [], []
    for _ in range(K):
        m = jnp.max(work, axis=-1, keepdims=True)
        eq = work == m
        idx = jnp.min(jnp.where(eq, jj, C), axis=-1, keepdims=True)
        vals_cols.append(m)
        idx_cols.append(idx)
        work = jnp.where(jj == idx, jnp.finfo(jnp.float32).min, work)
    vals_ref[...] = jnp.concatenate(vals_cols, axis=1)
    idx_ref[...] = jnp.concatenate(idx_cols, axis=1)


def _stage3(assignments, t):
    return pl.pallas_call(
        _adj_body,
        out_shape=[
            jax.ShapeDtypeStruct((G * C, K), jnp.float32),
            jax.ShapeDtypeStruct((G * C, K), jnp.int32),
        ],
    )(assignments, t)


def kernel(nodes, senders, receivers, n_node, n_edge, W1, b1, W2, b2):
    del n_node, n_edge
    assignments, coarse_nodes = _stage1(nodes, W1, b1.reshape(1, HID), W2,
                                        b2.reshape(1, C))
    zeros = jnp.zeros((SRW, C), jnp.float32)
    t = _stage2(assignments, senders, receivers, zeros)
    top_vals, top_idx = _stage3(assignments, t)
    batch_offset = jnp.arange(G, dtype=jnp.int32)[:, None] * C
    c_senders = (jnp.repeat(jnp.arange(C, dtype=jnp.int32), K)[None, :]
                 + batch_offset).reshape(-1)
    c_receivers = (top_idx.reshape(G, C * K) + batch_offset).reshape(-1)
    c_edge_weights = top_vals.reshape(-1, 1)
    return (coarse_nodes, c_senders, c_receivers, c_edge_weights,
            assignments)


# speedup vs baseline: 1.1859x; 1.1859x over previous
import functools

import jax
import jax.numpy as jnp
from jax import lax
from jax.experimental import pallas as pl
from jax.experimental.pallas import tpu as pltpu
from jax.experimental.pallas import tpu_sc as plsc

N = 10000
G = 8
NPG = N // G
E = 320000
D = 128
C = 64
K = 16
HID = 32

NC = 2
NS = 16
NW = NC * NS
EPW = E // NW
CH = 80
NCH = EPW // CH
NBUF = 5
PRE = 2
SRW = 624
TAIL = N - NS * SRW


def _mlp_body(x_ref, w1_ref, b1_ref, w2_ref, b2_ref, assign_ref, coarse_ref):
    w1 = w1_ref[...]
    b1 = b1_ref[0]
    w2 = w2_ref[...]
    b2 = b2_ref[0]
    for g in range(G):
        x = x_ref[pl.ds(g * NPG, NPG), :]
        h = jnp.dot(x, w1, preferred_element_type=jnp.float32)
        h = jnp.maximum(h + b1, 0.0)
        logits = jnp.dot(h, w2, preferred_element_type=jnp.float32) + b2
        m = jnp.max(logits, axis=-1, keepdims=True)
        e = jnp.exp(logits - m)
        a = e / jnp.sum(e, axis=-1, keepdims=True)
        assign_ref[pl.ds(g * NPG, NPG), :] = a
        coarse_ref[pl.ds(g * C, C), :] = lax.dot_general(
            a, x, (((0,), (0,)), ((), ())), preferred_element_type=jnp.float32)


def _stage1(nodes, w1, b1, w2, b2):
    return pl.pallas_call(
        _mlp_body,
        out_shape=[
            jax.ShapeDtypeStruct((N, C), jnp.float32),
            jax.ShapeDtypeStruct((G * C, D), jnp.float32),
        ],
    )(nodes, w1, b1, w2, b2)


def _edge_body(assign_hbm, senders_hbm, receivers_hbm, zeros_hbm, t_hbm,
               sidx_v, ridx_v, sbuf0, sbuf1, sbuf2, sbuf3, sbuf4,
               rows_v, t_sh, gsem, ssem):
    sbufs = (sbuf0, sbuf1, sbuf2, sbuf3, sbuf4)
    cid = lax.axis_index("c")
    sid = lax.axis_index("s")
    wid = sid * NC + cid
    stripe = pl.multiple_of(sid * SRW, 8)
    pltpu.sync_copy(zeros_hbm, t_sh.at[pl.ds(stripe, SRW)])

    @pl.when(sid == NS - 1)
    def _zero_tail():
        pltpu.sync_copy(zeros_hbm.at[pl.ds(0, TAIL)],
                        t_sh.at[pl.ds(NS * SRW, TAIL)])

    pltpu.sync_copy(senders_hbm.at[pl.ds(wid * EPW, EPW)], sidx_v)
    pltpu.sync_copy(receivers_hbm.at[pl.ds(wid * EPW, EPW)], ridx_v)
    plsc.subcore_barrier()

    for b in range(NBUF):
        pltpu.make_async_copy(assign_hbm.at[ridx_v.at[pl.ds(b * CH, CH)]],
                              rows_v.at[b], gsem.at[b]).start()

    def outer(g, carry):
        for b in range(NBUF):
            c = g * NBUF + b
            pltpu.make_async_copy(assign_hbm.at[ridx_v.at[pl.ds(b * CH, CH)]],
                                  rows_v.at[b], gsem.at[b]).wait()
            for j in range(CH // 16):
                sbufs[0][pl.ds(j * 16, 16)] = sidx_v[pl.ds(c * CH + j * 16, 16)]
            pltpu.sync_copy(rows_v.at[b], t_sh.at[sbufs[0]], add=True)
            cn = c + NBUF

            @pl.when(cn < NCH)
            def _prefetch():
                pltpu.make_async_copy(
                    assign_hbm.at[ridx_v.at[pl.ds(cn * CH, CH)]],
                    rows_v.at[b], gsem.at[b]).start()
        return carry

    lax.fori_loop(0, NCH // NBUF, outer, 0)
    plsc.subcore_barrier()
    pltpu.sync_copy(t_sh.at[pl.ds(stripe, SRW)],
                    t_hbm.at[cid, pl.ds(stripe, SRW)])

    @pl.when(sid == NS - 1)
    def _write_tail():
        pltpu.sync_copy(t_sh.at[pl.ds(NS * SRW, TAIL)],
                        t_hbm.at[cid, pl.ds(NS * SRW, TAIL)])


def _stage2(assignments, senders, receivers, zeros):
    run = pl.kernel(
        _edge_body,
        out_type=jax.ShapeDtypeStruct((NC, N, C), jnp.float32),
        mesh=plsc.VectorSubcoreMesh(core_axis_name="c", subcore_axis_name="s"),
        scratch_types=[
            pltpu.VMEM((EPW,), jnp.int32),
            pltpu.VMEM((EPW,), jnp.int32),
            pltpu.VMEM((CH,), jnp.int32),
            pltpu.VMEM((CH,), jnp.int32),
            pltpu.VMEM((CH,), jnp.int32),
            pltpu.VMEM((CH,), jnp.int32),
            pltpu.VMEM((CH,), jnp.int32),
            pltpu.VMEM((NBUF, CH, C), jnp.float32),
            pltpu.VMEM_SHARED((N, C), jnp.float32),
            pltpu.SemaphoreType.DMA((NBUF,)),
            pltpu.SemaphoreType.DMA((NBUF,)),
        ],
        compiler_params=pltpu.CompilerParams(use_tc_tiling_on_sc=False),
    )
    return run(assignments, senders, receivers, zeros)


def _adj_body(a_ref, t_ref, vals_ref, idx_ref):
    t = t_ref[0] + t_ref[1]
    a = a_ref[...]
    adjs = []
    for g in range(G):
        ag = a[g * NPG:(g + 1) * NPG]
        tg = t[g * NPG:(g + 1) * NPG]
        adjs.append(lax.dot_general(
            ag, tg, (((0,), (0,)), ((), ())),
            preferred_element_type=jnp.float32))
    work = jnp.concatenate(adjs, axis=0)
    jj = lax.broadcasted_iota(jnp.int32, (G * C, C), 1)
    vals_cols, idx_cols = [], []
    for _ in range(K):
        m = jnp.max(work, axis=-1, keepdims=True)
        eq = work == m
        idx = jnp.min(jnp.where(eq, jj, C), axis=-1, keepdims=True)
        vals_cols.append(m)
        idx_cols.append(idx)
        work = jnp.where(jj == idx, jnp.finfo(jnp.float32).min, work)
    vals_ref[...] = jnp.concatenate(vals_cols, axis=1)
    idx_ref[...] = jnp.concatenate(idx_cols, axis=1)


def _stage3(assignments, t):
    return pl.pallas_call(
        _adj_body,
        out_shape=[
            jax.ShapeDtypeStruct((G * C, K), jnp.float32),
            jax.ShapeDtypeStruct((G * C, K), jnp.int32),
        ],
    )(assignments, t)


def kernel(nodes, senders, receivers, n_node, n_edge, W1, b1, W2, b2):
    del n_node, n_edge
    assignments, coarse_nodes = _stage1(nodes, W1, b1.reshape(1, HID), W2,
                                        b2.reshape(1, C))
    zeros = jnp.zeros((SRW, C), jnp.float32)
    t = _stage2(assignments, senders, receivers, zeros)
    top_vals, top_idx = _stage3(assignments, t)
    batch_offset = jnp.arange(G, dtype=jnp.int32)[:, None] * C
    c_senders = (jnp.repeat(jnp.arange(C, dtype=jnp.int32), K)[None, :]
                 + batch_offset).reshape(-1)
    c_receivers = (top_idx.reshape(G, C * K) + batch_offset).reshape(-1)
    c_edge_weights = top_vals.reshape(-1, 1)
    return (coarse_nodes, c_senders, c_receivers, c_edge_weights,
            assignments)


# speedup vs baseline: 1.2332x; 1.0399x over previous
import functools

import jax
import jax.numpy as jnp
from jax import lax
from jax.experimental import pallas as pl
from jax.experimental.pallas import tpu as pltpu
from jax.experimental.pallas import tpu_sc as plsc

N = 10000
G = 8
NPG = N // G
E = 320000
D = 128
C = 64
K = 16
HID = 32

NC = 2
NS = 16
NW = NC * NS
EPW = E // NW
CH = 80
NCH = EPW // CH
NBUF = 5
PRE = 2
SRW = 624
TAIL = N - NS * SRW


def _mlp_body(x_ref, w1_ref, b1_ref, w2_ref, b2_ref, assign_ref, coarse_ref):
    x = x_ref[...]
    h = jnp.dot(x, w1_ref[...], preferred_element_type=jnp.float32)
    h = jnp.maximum(h + b1_ref[0], 0.0)
    logits = jnp.dot(h, w2_ref[...], preferred_element_type=jnp.float32)
    logits = logits + b2_ref[0]
    m = jnp.max(logits, axis=-1, keepdims=True)
    e = jnp.exp(logits - m)
    a = e / jnp.sum(e, axis=-1, keepdims=True)
    assign_ref[...] = a
    for g in range(G):
        coarse_ref[pl.ds(g * C, C), :] = lax.dot_general(
            a[g * NPG:(g + 1) * NPG], x[g * NPG:(g + 1) * NPG],
            (((0,), (0,)), ((), ())), preferred_element_type=jnp.float32)


def _stage1(nodes, w1, b1, w2, b2):
    return pl.pallas_call(
        _mlp_body,
        out_shape=[
            jax.ShapeDtypeStruct((N, C), jnp.float32),
            jax.ShapeDtypeStruct((G * C, D), jnp.float32),
        ],
        compiler_params=pltpu.CompilerParams(fuse_transposed_lhs_in_matmul=True),
    )(nodes, w1, b1, w2, b2)


def _edge_body(assign_hbm, senders_hbm, receivers_hbm, zeros_hbm, t_hbm,
               sidx_v, ridx_v, sbuf0, sbuf1, sbuf2, sbuf3, sbuf4,
               rows_v, t_sh, gsem, ssem):
    sbufs = (sbuf0, sbuf1, sbuf2, sbuf3, sbuf4)
    cid = lax.axis_index("c")
    sid = lax.axis_index("s")
    wid = sid * NC + cid
    stripe = pl.multiple_of(sid * SRW, 8)
    pltpu.sync_copy(zeros_hbm, t_sh.at[pl.ds(stripe, SRW)])

    @pl.when(sid == NS - 1)
    def _zero_tail():
        pltpu.sync_copy(zeros_hbm.at[pl.ds(0, TAIL)],
                        t_sh.at[pl.ds(NS * SRW, TAIL)])

    pltpu.sync_copy(senders_hbm.at[pl.ds(wid * EPW, EPW)], sidx_v)
    pltpu.sync_copy(receivers_hbm.at[pl.ds(wid * EPW, EPW)], ridx_v)
    plsc.subcore_barrier()

    for b in range(NBUF):
        pltpu.make_async_copy(assign_hbm.at[ridx_v.at[pl.ds(b * CH, CH)]],
                              rows_v.at[b], gsem.at[b]).start()

    def outer(g, carry):
        for b in range(NBUF):
            c = g * NBUF + b
            pltpu.make_async_copy(assign_hbm.at[ridx_v.at[pl.ds(b * CH, CH)]],
                                  rows_v.at[b], gsem.at[b]).wait()
            for j in range(CH // 16):
                sbufs[0][pl.ds(j * 16, 16)] = sidx_v[pl.ds(c * CH + j * 16, 16)]
            pltpu.sync_copy(rows_v.at[b], t_sh.at[sbufs[0]], add=True)
            cn = c + NBUF

            @pl.when(cn < NCH)
            def _prefetch():
                pltpu.make_async_copy(
                    assign_hbm.at[ridx_v.at[pl.ds(cn * CH, CH)]],
                    rows_v.at[b], gsem.at[b]).start()
        return carry

    lax.fori_loop(0, NCH // NBUF, outer, 0)
    plsc.subcore_barrier()
    pltpu.sync_copy(t_sh.at[pl.ds(stripe, SRW)],
                    t_hbm.at[cid, pl.ds(stripe, SRW)])

    @pl.when(sid == NS - 1)
    def _write_tail():
        pltpu.sync_copy(t_sh.at[pl.ds(NS * SRW, TAIL)],
                        t_hbm.at[cid, pl.ds(NS * SRW, TAIL)])


def _stage2(assignments, senders, receivers, zeros):
    run = pl.kernel(
        _edge_body,
        out_type=jax.ShapeDtypeStruct((NC, N, C), jnp.float32),
        mesh=plsc.VectorSubcoreMesh(core_axis_name="c", subcore_axis_name="s"),
        scratch_types=[
            pltpu.VMEM((EPW,), jnp.int32),
            pltpu.VMEM((EPW,), jnp.int32),
            pltpu.VMEM((CH,), jnp.int32),
            pltpu.VMEM((CH,), jnp.int32),
            pltpu.VMEM((CH,), jnp.int32),
            pltpu.VMEM((CH,), jnp.int32),
            pltpu.VMEM((CH,), jnp.int32),
            pltpu.VMEM((NBUF, CH, C), jnp.float32),
            pltpu.VMEM_SHARED((N, C), jnp.float32),
            pltpu.SemaphoreType.DMA((NBUF,)),
            pltpu.SemaphoreType.DMA((NBUF,)),
        ],
        compiler_params=pltpu.CompilerParams(use_tc_tiling_on_sc=False),
    )
    return run(assignments, senders, receivers, zeros)


def _adj_body(a_ref, t_ref, vals_ref, idx_ref):
    t = t_ref[0] + t_ref[1]
    a = a_ref[...]
    adjs = []
    for g in range(G):
        ag = a[g * NPG:(g + 1) * NPG]
        tg = t[g * NPG:(g + 1) * NPG]
        adjs.append(lax.dot_general(
            ag, tg, (((0,), (0,)), ((), ())),
            preferred_element_type=jnp.float32))
    work = jnp.concatenate(adjs, axis=0)
    jj = lax.broadcasted_iota(jnp.int32, (G * C, C), 1)
    vals_cols, idx_cols = [], []
    for _ in range(K):
        m = jnp.max(work, axis=-1, keepdims=True)
        eq = work == m
        idx = jnp.min(jnp.where(eq, jj, C), axis=-1, keepdims=True)
        vals_cols.append(m)
        idx_cols.append(idx)
        work = jnp.where(jj == idx, jnp.finfo(jnp.float32).min, work)
    vals_ref[...] = jnp.concatenate(vals_cols, axis=1)
    idx_ref[...] = jnp.concatenate(idx_cols, axis=1)


def _stage3(assignments, t):
    return pl.pallas_call(
        _adj_body,
        out_shape=[
            jax.ShapeDtypeStruct((G * C, K), jnp.float32),
            jax.ShapeDtypeStruct((G * C, K), jnp.int32),
        ],
        compiler_params=pltpu.CompilerParams(fuse_transposed_lhs_in_matmul=True),
    )(assignments, t)


def kernel(nodes, senders, receivers, n_node, n_edge, W1, b1, W2, b2):
    del n_node, n_edge
    assignments, coarse_nodes = _stage1(nodes, W1, b1.reshape(1, HID), W2,
                                        b2.reshape(1, C))
    zeros = jnp.zeros((SRW, C), jnp.float32)
    t = _stage2(assignments, senders, receivers, zeros)
    top_vals, top_idx = _stage3(assignments, t)
    batch_offset = jnp.arange(G, dtype=jnp.int32)[:, None] * C
    c_senders = (jnp.repeat(jnp.arange(C, dtype=jnp.int32), K)[None, :]
                 + batch_offset).reshape(-1)
    c_receivers = (top_idx.reshape(G, C * K) + batch_offset).reshape(-1)
    c_edge_weights = top_vals.reshape(-1, 1)
    return (coarse_nodes, c_senders, c_receivers, c_edge_weights,
            assignments)


# speedup vs baseline: 1.2814x; 1.0391x over previous
import functools

import jax
import jax.numpy as jnp
from jax import lax
from jax.experimental import pallas as pl
from jax.experimental.pallas import tpu as pltpu
from jax.experimental.pallas import tpu_sc as plsc

N = 10000
G = 8
NPG = N // G
E = 320000
D = 128
C = 64
K = 16
HID = 32

NC = 2
NS = 16
NW = NC * NS
EPW = E // NW
CH = 80
NCH = EPW // CH
NBUF = 5
PRE = 2
SRW = 624
TAIL = N - NS * SRW


def _mlp_body(x_ref, w1_ref, b1_ref, w2_ref, b2_ref, assign_ref, coarse_ref):
    x = x_ref[...]
    h = jnp.dot(x, w1_ref[...], preferred_element_type=jnp.float32)
    h = jnp.maximum(h + b1_ref[0], 0.0)
    logits = jnp.dot(h, w2_ref[...], preferred_element_type=jnp.float32)
    logits = logits + b2_ref[0]
    m = jnp.max(logits, axis=-1, keepdims=True)
    e = jnp.exp(logits - m)
    a = e / jnp.sum(e, axis=-1, keepdims=True)
    assign_ref[...] = a
    for g in range(G):
        coarse_ref[pl.ds(g * C, C), :] = lax.dot_general(
            a[g * NPG:(g + 1) * NPG], x[g * NPG:(g + 1) * NPG],
            (((0,), (0,)), ((), ())), preferred_element_type=jnp.float32)


def _stage1(nodes, w1, b1, w2, b2):
    return pl.pallas_call(
        _mlp_body,
        out_shape=[
            jax.ShapeDtypeStruct((N, C), jnp.float32),
            jax.ShapeDtypeStruct((G * C, D), jnp.float32),
        ],
        compiler_params=pltpu.CompilerParams(fuse_transposed_lhs_in_matmul=True),
    )(nodes, w1, b1, w2, b2)


def _edge_body(assign_hbm, senders_hbm, receivers_hbm, zeros_hbm, t_hbm,
               sidx_v, ridx_v, sbuf0, sbuf1, sbuf2, sbuf3, sbuf4,
               rows_v, t_sh, gsem, ssem):
    sbufs = (sbuf0, sbuf1, sbuf2, sbuf3, sbuf4)
    cid = lax.axis_index("c")
    sid = lax.axis_index("s")
    wid = sid * NC + cid
    pltpu.sync_copy(receivers_hbm.at[pl.ds(wid * EPW, EPW)], ridx_v)

    for b in range(NBUF):
        pltpu.make_async_copy(assign_hbm.at[ridx_v.at[pl.ds(b * CH, CH)]],
                              rows_v.at[b], gsem.at[b]).start()

    pltpu.sync_copy(senders_hbm.at[pl.ds(wid * EPW, EPW)], sidx_v)
    stripe = pl.multiple_of(sid * SRW, 8)
    pltpu.sync_copy(zeros_hbm, t_sh.at[pl.ds(stripe, SRW)])

    @pl.when(sid == NS - 1)
    def _zero_tail():
        pltpu.sync_copy(zeros_hbm.at[pl.ds(0, TAIL)],
                        t_sh.at[pl.ds(NS * SRW, TAIL)])

    plsc.subcore_barrier()

    def outer(g, carry):
        for b in range(NBUF):
            c = g * NBUF + b
            pltpu.make_async_copy(assign_hbm.at[ridx_v.at[pl.ds(b * CH, CH)]],
                                  rows_v.at[b], gsem.at[b]).wait()
            for j in range(CH // 16):
                sbufs[0][pl.ds(j * 16, 16)] = sidx_v[pl.ds(c * CH + j * 16, 16)]
            pltpu.sync_copy(rows_v.at[b], t_sh.at[sbufs[0]], add=True)
            cn = c + NBUF

            @pl.when(cn < NCH)
            def _prefetch():
                pltpu.make_async_copy(
                    assign_hbm.at[ridx_v.at[pl.ds(cn * CH, CH)]],
                    rows_v.at[b], gsem.at[b]).start()
        return carry

    lax.fori_loop(0, NCH // NBUF, outer, 0)
    plsc.subcore_barrier()
    pltpu.sync_copy(t_sh.at[pl.ds(stripe, SRW)],
                    t_hbm.at[cid, pl.ds(stripe, SRW)])

    @pl.when(sid == NS - 1)
    def _write_tail():
        pltpu.sync_copy(t_sh.at[pl.ds(NS * SRW, TAIL)],
                        t_hbm.at[cid, pl.ds(NS * SRW, TAIL)])


def _stage2(assignments, senders, receivers, zeros):
    run = pl.kernel(
        _edge_body,
        out_type=jax.ShapeDtypeStruct((NC, N, C), jnp.float32),
        mesh=plsc.VectorSubcoreMesh(core_axis_name="c", subcore_axis_name="s"),
        scratch_types=[
            pltpu.VMEM((EPW,), jnp.int32),
            pltpu.VMEM((EPW,), jnp.int32),
            pltpu.VMEM((CH,), jnp.int32),
            pltpu.VMEM((CH,), jnp.int32),
            pltpu.VMEM((CH,), jnp.int32),
            pltpu.VMEM((CH,), jnp.int32),
            pltpu.VMEM((CH,), jnp.int32),
            pltpu.VMEM((NBUF, CH, C), jnp.float32),
            pltpu.VMEM_SHARED((N, C), jnp.float32),
            pltpu.SemaphoreType.DMA((NBUF,)),
            pltpu.SemaphoreType.DMA((NBUF,)),
        ],
        compiler_params=pltpu.CompilerParams(use_tc_tiling_on_sc=False),
    )
    return run(assignments, senders, receivers, zeros)


def _adj_body(a_ref, t_ref, vals_ref, idx_ref):
    t = t_ref[0] + t_ref[1]
    a = a_ref[...]
    adjs = []
    for g in range(G):
        ag = a[g * NPG:(g + 1) * NPG]
        tg = t[g * NPG:(g + 1) * NPG]
        adjs.append(lax.dot_general(
            ag, tg, (((0,), (0,)), ((), ())),
            preferred_element_type=jnp.float32))
    work = jnp.concatenate(adjs, axis=0)
    jj = lax.broadcasted_iota(jnp.int32, (G * C, C), 1)
    vals_cols, idx_cols = [], []
    for _ in range(K):
        m = jnp.max(work, axis=-1, keepdims=True)
        idx = jnp.argmax(work, axis=-1).astype(jnp.int32)[:, None]
        vals_cols.append(m)
        idx_cols.append(idx)
        work = jnp.where(jj == idx, jnp.finfo(jnp.float32).min, work)
    vals_ref[...] = jnp.concatenate(vals_cols, axis=1)
    idx_ref[...] = jnp.concatenate(idx_cols, axis=1)


def _stage3(assignments, t):
    return pl.pallas_call(
        _adj_body,
        out_shape=[
            jax.ShapeDtypeStruct((G * C, K), jnp.float32),
            jax.ShapeDtypeStruct((G * C, K), jnp.int32),
        ],
        compiler_params=pltpu.CompilerParams(fuse_transposed_lhs_in_matmul=True),
    )(assignments, t)


def kernel(nodes, senders, receivers, n_node, n_edge, W1, b1, W2, b2):
    del n_node, n_edge
    assignments, coarse_nodes = _stage1(nodes, W1, b1.reshape(1, HID), W2,
                                        b2.reshape(1, C))
    zeros = jnp.zeros((SRW, C), jnp.float32)
    t = _stage2(assignments, senders, receivers, zeros)
    top_vals, top_idx = _stage3(assignments, t)
    batch_offset = jnp.arange(G, dtype=jnp.int32)[:, None] * C
    c_senders = (jnp.repeat(jnp.arange(C, dtype=jnp.int32), K)[None, :]
                 + batch_offset).reshape(-1)
    c_receivers = (top_idx.reshape(G, C * K) + batch_offset).reshape(-1)
    c_edge_weights = top_vals.reshape(-1, 1)
    return (coarse_nodes, c_senders, c_receivers, c_edge_weights,
            assignments)


# speedup vs baseline: 1.3958x; 1.0892x over previous
import functools

import jax
import jax.numpy as jnp
from jax import lax
from jax.experimental import pallas as pl
from jax.experimental.pallas import tpu as pltpu
from jax.experimental.pallas import tpu_sc as plsc

N = 10000
G = 8
NPG = N // G
E = 320000
D = 128
C = 64
K = 16
HID = 32

NC = 2
NS = 16
NW = NC * NS
EPW = E // NW
CH = 80
NCH = EPW // CH
NBUF = 5
PRE = 2
SRW = 624
TAIL = N - NS * SRW


def _mlp_body(x_ref, w1_ref, b1_ref, w2_ref, b2_ref, assign_ref, coarse_ref):
    x = x_ref[...]
    h = jnp.dot(x, w1_ref[...], preferred_element_type=jnp.float32)
    h = jnp.maximum(h + b1_ref[0], 0.0)
    logits = jnp.dot(h, w2_ref[...], preferred_element_type=jnp.float32)
    logits = logits + b2_ref[0]
    m = jnp.max(logits, axis=-1, keepdims=True)
    e = jnp.exp(logits - m)
    a = e / jnp.sum(e, axis=-1, keepdims=True)
    assign_ref[...] = a
    for g in range(G):
        coarse_ref[pl.ds(g * C, C), :] = lax.dot_general(
            a[g * NPG:(g + 1) * NPG], x[g * NPG:(g + 1) * NPG],
            (((0,), (0,)), ((), ())), preferred_element_type=jnp.float32)


def _stage1(nodes, w1, b1, w2, b2):
    return pl.pallas_call(
        _mlp_body,
        out_shape=[
            jax.ShapeDtypeStruct((N, C), jnp.float32),
            jax.ShapeDtypeStruct((G * C, D), jnp.float32),
        ],
        compiler_params=pltpu.CompilerParams(fuse_transposed_lhs_in_matmul=True),
    )(nodes, w1, b1, w2, b2)


def _edge_body(assign_hbm, senders_hbm, receivers_hbm, zeros_hbm, t_hbm,
               sidx_v, ridx_v, sbuf0, sbuf1, sbuf2, sbuf3, sbuf4,
               rows_v, t_sh, gsem, ssem):
    sbufs = (sbuf0, sbuf1, sbuf2, sbuf3, sbuf4)
    cid = lax.axis_index("c")
    sid = lax.axis_index("s")
    wid = sid * NC + cid
    pltpu.sync_copy(receivers_hbm.at[pl.ds(wid * EPW, EPW)], ridx_v)

    for b in range(NBUF):
        pltpu.make_async_copy(assign_hbm.at[ridx_v.at[pl.ds(b * CH, CH)]],
                              rows_v.at[b], gsem.at[b]).start()

    pltpu.sync_copy(senders_hbm.at[pl.ds(wid * EPW, EPW)], sidx_v)
    stripe = pl.multiple_of(sid * SRW, 8)
    pltpu.sync_copy(zeros_hbm, t_sh.at[pl.ds(stripe, SRW)])

    @pl.when(sid == NS - 1)
    def _zero_tail():
        pltpu.sync_copy(zeros_hbm.at[pl.ds(0, TAIL)],
                        t_sh.at[pl.ds(NS * SRW, TAIL)])

    plsc.subcore_barrier()

    def outer(g, carry):
        for b in range(NBUF):
            c = g * NBUF + b
            pltpu.make_async_copy(assign_hbm.at[ridx_v.at[pl.ds(b * CH, CH)]],
                                  rows_v.at[b], gsem.at[b]).wait()
            for j in range(CH // 16):
                sbufs[0][pl.ds(j * 16, 16)] = sidx_v[pl.ds(c * CH + j * 16, 16)]
            pltpu.sync_copy(rows_v.at[b], t_sh.at[sbufs[0]], add=True)
            cn = c + NBUF

            @pl.when(cn < NCH)
            def _prefetch():
                pltpu.make_async_copy(
                    assign_hbm.at[ridx_v.at[pl.ds(cn * CH, CH)]],
                    rows_v.at[b], gsem.at[b]).start()
        return carry

    lax.fori_loop(0, NCH // NBUF, outer, 0)
    plsc.subcore_barrier()
    pltpu.sync_copy(t_sh.at[pl.ds(stripe, SRW)],
                    t_hbm.at[cid, pl.ds(stripe, SRW), pl.ds(0, C)])

    @pl.when(sid == NS - 1)
    def _write_tail():
        pltpu.sync_copy(t_sh.at[pl.ds(NS * SRW, TAIL)],
                        t_hbm.at[cid, pl.ds(NS * SRW, TAIL), pl.ds(0, C)])


def _stage2(assignments, senders, receivers, zeros):
    run = pl.kernel(
        _edge_body,
        out_type=jax.ShapeDtypeStruct((NC, N, 2 * C), jnp.float32),
        mesh=plsc.VectorSubcoreMesh(core_axis_name="c", subcore_axis_name="s"),
        scratch_types=[
            pltpu.VMEM((EPW,), jnp.int32),
            pltpu.VMEM((EPW,), jnp.int32),
            pltpu.VMEM((CH,), jnp.int32),
            pltpu.VMEM((CH,), jnp.int32),
            pltpu.VMEM((CH,), jnp.int32),
            pltpu.VMEM((CH,), jnp.int32),
            pltpu.VMEM((CH,), jnp.int32),
            pltpu.VMEM((NBUF, CH, C), jnp.float32),
            pltpu.VMEM_SHARED((N, C), jnp.float32),
            pltpu.SemaphoreType.DMA((NBUF,)),
            pltpu.SemaphoreType.DMA((NBUF,)),
        ],
        compiler_params=pltpu.CompilerParams(use_tc_tiling_on_sc=False),
    )
    return run(assignments, senders, receivers, zeros)


def _adj_body(a_ref, t_ref, vals_ref, idx_ref):
    t = t_ref[0, :, :C] + t_ref[1, :, :C]
    a = a_ref[...]
    adjs = []
    for g in range(G):
        ag = a[g * NPG:(g + 1) * NPG]
        tg = t[g * NPG:(g + 1) * NPG]
        adjs.append(lax.dot_general(
            ag, tg, (((0,), (0,)), ((), ())),
            preferred_element_type=jnp.float32))
    work = jnp.concatenate(adjs, axis=0)
    jj = lax.broadcasted_iota(jnp.int32, (G * C, C), 1)
    vals_cols, idx_cols = [], []
    for _ in range(K):
        m = jnp.max(work, axis=-1, keepdims=True)
        idx = jnp.argmax(work, axis=-1).astype(jnp.int32)[:, None]
        vals_cols.append(m)
        idx_cols.append(idx)
        work = jnp.where(jj == idx, jnp.finfo(jnp.float32).min, work)
    vals_ref[...] = jnp.concatenate(vals_cols, axis=1)
    idx_ref[...] = jnp.concatenate(idx_cols, axis=1)


def _stage3(assignments, t):
    return pl.pallas_call(
        _adj_body,
        out_shape=[
            jax.ShapeDtypeStruct((G * C, K), jnp.float32),
            jax.ShapeDtypeStruct((G * C, K), jnp.int32),
        ],
        compiler_params=pltpu.CompilerParams(fuse_transposed_lhs_in_matmul=True),
    )(assignments, t)


def kernel(nodes, senders, receivers, n_node, n_edge, W1, b1, W2, b2):
    del n_node, n_edge
    assignments, coarse_nodes = _stage1(nodes, W1, b1.reshape(1, HID), W2,
                                        b2.reshape(1, C))
    zeros = jnp.zeros((SRW, C), jnp.float32)
    t = _stage2(assignments, senders, receivers, zeros)
    top_vals, top_idx = _stage3(assignments, t)
    batch_offset = jnp.arange(G, dtype=jnp.int32)[:, None] * C
    c_senders = (jnp.repeat(jnp.arange(C, dtype=jnp.int32), K)[None, :]
                 + batch_offset).reshape(-1)
    c_receivers = (top_idx.reshape(G, C * K) + batch_offset).reshape(-1)
    c_edge_weights = top_vals.reshape(-1, 1)
    return (coarse_nodes, c_senders, c_receivers, c_edge_weights,
            assignments)


# speedup vs baseline: 1.4295x; 1.0242x over previous
import functools

import jax
import jax.numpy as jnp
from jax import lax
from jax.experimental import pallas as pl
from jax.experimental.pallas import tpu as pltpu
from jax.experimental.pallas import tpu_sc as plsc

N = 10000
G = 8
NPG = N // G
E = 320000
D = 128
C = 64
K = 16
HID = 32

NC = 2
NS = 16
NW = NC * NS
EPW = E // NW
CH = 80
NCH = EPW // CH
NBUF = 5
PRE = 2
SRW = 624
TAIL = N - NS * SRW


def _mlp_body(x_ref, w1_ref, b1_ref, w2_ref, b2_ref, assign_ref, apack_ref,
              coarse_ref):
    x = x_ref[...]
    h = jnp.dot(x, w1_ref[...], preferred_element_type=jnp.float32)
    h = jnp.maximum(h + b1_ref[0], 0.0)
    logits = jnp.dot(h, w2_ref[...], preferred_element_type=jnp.float32)
    logits = logits + b2_ref[0]
    m = jnp.max(logits, axis=-1, keepdims=True)
    e = jnp.exp(logits - m)
    a = e / jnp.sum(e, axis=-1, keepdims=True)
    assign_ref[...] = a
    apack_ref[...] = jnp.concatenate([a[:N // 2], a[N // 2:]], axis=1)
    for g in range(G):
        coarse_ref[pl.ds(g * C, C), :] = lax.dot_general(
            a[g * NPG:(g + 1) * NPG], x[g * NPG:(g + 1) * NPG],
            (((0,), (0,)), ((), ())), preferred_element_type=jnp.float32)


def _stage1(nodes, w1, b1, w2, b2):
    return pl.pallas_call(
        _mlp_body,
        out_shape=[
            jax.ShapeDtypeStruct((N, C), jnp.float32),
            jax.ShapeDtypeStruct((N // 2, 2 * C), jnp.float32),
            jax.ShapeDtypeStruct((G * C, D), jnp.float32),
        ],
        compiler_params=pltpu.CompilerParams(fuse_transposed_lhs_in_matmul=True),
    )(nodes, w1, b1, w2, b2)


def _edge_body(assign_hbm, senders_hbm, receivers_hbm, zeros_hbm, t_hbm,
               sidx_v, ridx_v, sbuf0, sbuf1, sbuf2, sbuf3, sbuf4,
               rows_v, t_sh, gsem, ssem):
    sbufs = (sbuf0, sbuf1, sbuf2, sbuf3, sbuf4)
    cid = lax.axis_index("c")
    sid = lax.axis_index("s")
    wid = sid * NC + cid
    pltpu.sync_copy(receivers_hbm.at[pl.ds(wid * EPW, EPW)], ridx_v)

    for b in range(NBUF):
        pltpu.make_async_copy(assign_hbm.at[ridx_v.at[pl.ds(b * CH, CH)]],
                              rows_v.at[b], gsem.at[b]).start()

    pltpu.sync_copy(senders_hbm.at[pl.ds(wid * EPW, EPW)], sidx_v)
    stripe = pl.multiple_of(sid * SRW, 8)
    pltpu.sync_copy(zeros_hbm, t_sh.at[pl.ds(stripe, SRW)])

    @pl.when(sid == NS - 1)
    def _zero_tail():
        pltpu.sync_copy(zeros_hbm.at[pl.ds(0, TAIL)],
                        t_sh.at[pl.ds(NS * SRW, TAIL)])

    plsc.subcore_barrier()

    def outer(g, carry):
        for b in range(NBUF):
            c = g * NBUF + b
            pltpu.make_async_copy(assign_hbm.at[ridx_v.at[pl.ds(b * CH, CH)]],
                                  rows_v.at[b], gsem.at[b]).wait()
            for j in range(CH // 16):
                sbufs[0][pl.ds(j * 16, 16)] = sidx_v[pl.ds(c * CH + j * 16, 16)]
            pltpu.sync_copy(rows_v.at[b], t_sh.at[sbufs[0]], add=True)
            cn = c + NBUF

            @pl.when(cn < NCH)
            def _prefetch():
                pltpu.make_async_copy(
                    assign_hbm.at[ridx_v.at[pl.ds(cn * CH, CH)]],
                    rows_v.at[b], gsem.at[b]).start()
        return carry

    lax.fori_loop(0, NCH // NBUF, outer, 0)
    plsc.subcore_barrier()
    pltpu.sync_copy(t_sh.at[pl.ds(stripe, SRW)],
                    t_hbm.at[cid, pl.ds(stripe, SRW), pl.ds(0, C)])

    @pl.when(sid == NS - 1)
    def _write_tail():
        pltpu.sync_copy(t_sh.at[pl.ds(NS * SRW, TAIL)],
                        t_hbm.at[cid, pl.ds(NS * SRW, TAIL), pl.ds(0, C)])


def _stage2(assignments, senders, receivers, zeros):
    run = pl.kernel(
        _edge_body,
        out_type=jax.ShapeDtypeStruct((NC, N, 2 * C), jnp.float32),
        mesh=plsc.VectorSubcoreMesh(core_axis_name="c", subcore_axis_name="s"),
        scratch_types=[
            pltpu.VMEM((EPW,), jnp.int32),
            pltpu.VMEM((EPW,), jnp.int32),
            pltpu.VMEM((CH,), jnp.int32),
            pltpu.VMEM((CH,), jnp.int32),
            pltpu.VMEM((CH,), jnp.int32),
            pltpu.VMEM((CH,), jnp.int32),
            pltpu.VMEM((CH,), jnp.int32),
            pltpu.VMEM((NBUF, CH, C), jnp.float32),
            pltpu.VMEM_SHARED((N, C), jnp.float32),
            pltpu.SemaphoreType.DMA((NBUF,)),
            pltpu.SemaphoreType.DMA((NBUF,)),
        ],
        compiler_params=pltpu.CompilerParams(use_tc_tiling_on_sc=False),
    )
    return run(assignments, senders, receivers, zeros)


def _adj_body(a_ref, t_ref, vals_ref, idx_ref):
    t = t_ref[0, :, :C] + t_ref[1, :, :C]
    a = a_ref[...]
    adjs = []
    for g in range(G):
        ag = a[g * NPG:(g + 1) * NPG]
        tg = t[g * NPG:(g + 1) * NPG]
        adjs.append(lax.dot_general(
            ag, tg, (((0,), (0,)), ((), ())),
            preferred_element_type=jnp.float32))
    work = jnp.concatenate(adjs, axis=0)
    jj = lax.broadcasted_iota(jnp.int32, (G * C, C), 1)
    vals_cols, idx_cols = [], []
    for _ in range(K):
        m = jnp.max(work, axis=-1, keepdims=True)
        idx = jnp.argmax(work, axis=-1).astype(jnp.int32)[:, None]
        vals_cols.append(m)
        idx_cols.append(idx)
        work = jnp.where(jj == idx, jnp.finfo(jnp.float32).min, work)
    vals_ref[...] = jnp.concatenate(vals_cols, axis=1)
    idx_ref[...] = jnp.concatenate(idx_cols, axis=1)


def _stage3(assignments, t):
    return pl.pallas_call(
        _adj_body,
        out_shape=[
            jax.ShapeDtypeStruct((G * C, K), jnp.float32),
            jax.ShapeDtypeStruct((G * C, K), jnp.int32),
        ],
        compiler_params=pltpu.CompilerParams(fuse_transposed_lhs_in_matmul=True),
    )(assignments, t)


def kernel(nodes, senders, receivers, n_node, n_edge, W1, b1, W2, b2):
    del n_node, n_edge
    assignments, apack, coarse_nodes = _stage1(nodes, W1, b1.reshape(1, HID),
                                               W2, b2.reshape(1, C))
    zeros = jnp.zeros((SRW, C), jnp.float32)
    recv_q = jnp.where(receivers < N // 2, 2 * receivers,
                       2 * (receivers - N // 2) + 1)
    t = _stage2(apack.reshape(N, C), senders, recv_q, zeros)
    top_vals, top_idx = _stage3(assignments, t)
    batch_offset = jnp.arange(G, dtype=jnp.int32)[:, None] * C
    c_senders = (jnp.repeat(jnp.arange(C, dtype=jnp.int32), K)[None, :]
                 + batch_offset).reshape(-1)
    c_receivers = (top_idx.reshape(G, C * K) + batch_offset).reshape(-1)
    c_edge_weights = top_vals.reshape(-1, 1)
    return (coarse_nodes, c_senders, c_receivers, c_edge_weights,
            assignments)


# speedup vs baseline: 1.4613x; 1.0222x over previous
import functools

import jax
import jax.numpy as jnp
from jax import lax
from jax.experimental import pallas as pl
from jax.experimental.pallas import tpu as pltpu
from jax.experimental.pallas import tpu_sc as plsc

N = 10000
G = 8
NPG = N // G
E = 320000
D = 128
C = 64
K = 16
HID = 32

NC = 2
NS = 16
NW = NC * NS
EPW = E // NW
CH = 80
NCH = EPW // CH
NBUF = 5
PRE = 2
SRW = 624
TAIL = N - NS * SRW


def _mlp_body(x_ref, w1_ref, b1_ref, w2_ref, b2_ref, recv_ref, assign_ref,
              apack_ref, coarse_ref, recvq_ref):
    r = recv_ref[...]
    recvq_ref[...] = jnp.where(r < N // 2, 2 * r, 2 * (r - N // 2) + 1)
    x = x_ref[...]
    h = jnp.dot(x, w1_ref[...], preferred_element_type=jnp.float32)
    h = jnp.maximum(h + b1_ref[0], 0.0)
    logits = jnp.dot(h, w2_ref[...], preferred_element_type=jnp.float32)
    logits = logits + b2_ref[0]
    m = jnp.max(logits, axis=-1, keepdims=True)
    e = jnp.exp(logits - m)
    a = e / jnp.sum(e, axis=-1, keepdims=True)
    assign_ref[...] = a
    apack_ref[...] = jnp.concatenate([a[:N // 2], a[N // 2:]], axis=1)
    for g in range(G):
        coarse_ref[pl.ds(g * C, C), :] = lax.dot_general(
            a[g * NPG:(g + 1) * NPG], x[g * NPG:(g + 1) * NPG],
            (((0,), (0,)), ((), ())), preferred_element_type=jnp.float32)


def _stage1(nodes, w1, b1, w2, b2, receivers):
    return pl.pallas_call(
        _mlp_body,
        out_shape=[
            jax.ShapeDtypeStruct((N, C), jnp.float32),
            jax.ShapeDtypeStruct((N // 2, 2 * C), jnp.float32),
            jax.ShapeDtypeStruct((G * C, D), jnp.float32),
            jax.ShapeDtypeStruct((E,), jnp.int32),
        ],
        compiler_params=pltpu.CompilerParams(fuse_transposed_lhs_in_matmul=True),
    )(nodes, w1, b1, w2, b2, receivers)


def _edge_body(assign_hbm, senders_hbm, receivers_hbm, zeros_hbm, t_hbm,
               sidx_v, ridx_v, sbuf0, sbuf1, sbuf2, sbuf3, sbuf4,
               rows_v, t_sh, gsem, ssem):
    sbufs = (sbuf0, sbuf1, sbuf2, sbuf3, sbuf4)
    cid = lax.axis_index("c")
    sid = lax.axis_index("s")
    wid = sid * NC + cid
    pltpu.sync_copy(receivers_hbm.at[pl.ds(wid * EPW, EPW)], ridx_v)

    for b in range(NBUF):
        pltpu.make_async_copy(assign_hbm.at[ridx_v.at[pl.ds(b * CH, CH)]],
                              rows_v.at[b], gsem.at[b]).start()

    pltpu.sync_copy(senders_hbm.at[pl.ds(wid * EPW, EPW)], sidx_v)
    stripe = pl.multiple_of(sid * SRW, 8)
    pltpu.sync_copy(zeros_hbm, t_sh.at[pl.ds(stripe, SRW)])

    @pl.when(sid == NS - 1)
    def _zero_tail():
        pltpu.sync_copy(zeros_hbm.at[pl.ds(0, TAIL)],
                        t_sh.at[pl.ds(NS * SRW, TAIL)])

    plsc.subcore_barrier()

    def outer(g, carry):
        for b in range(NBUF):
            c = g * NBUF + b
            pltpu.make_async_copy(assign_hbm.at[ridx_v.at[pl.ds(b * CH, CH)]],
                                  rows_v.at[b], gsem.at[b]).wait()
            for j in range(CH // 16):
                sbufs[0][pl.ds(j * 16, 16)] = sidx_v[pl.ds(c * CH + j * 16, 16)]
            pltpu.sync_copy(rows_v.at[b], t_sh.at[sbufs[0]], add=True)
            cn = c + NBUF

            @pl.when(cn < NCH)
            def _prefetch():
                pltpu.make_async_copy(
                    assign_hbm.at[ridx_v.at[pl.ds(cn * CH, CH)]],
                    rows_v.at[b], gsem.at[b]).start()
        return carry

    lax.fori_loop(0, NCH // NBUF, outer, 0)
    plsc.subcore_barrier()
    pltpu.sync_copy(t_sh.at[pl.ds(stripe, SRW)],
                    t_hbm.at[cid, pl.ds(stripe, SRW), pl.ds(0, C)])

    @pl.when(sid == NS - 1)
    def _write_tail():
        pltpu.sync_copy(t_sh.at[pl.ds(NS * SRW, TAIL)],
                        t_hbm.at[cid, pl.ds(NS * SRW, TAIL), pl.ds(0, C)])


def _stage2(assignments, senders, receivers, zeros):
    run = pl.kernel(
        _edge_body,
        out_type=jax.ShapeDtypeStruct((NC, N, 2 * C), jnp.float32),
        mesh=plsc.VectorSubcoreMesh(core_axis_name="c", subcore_axis_name="s"),
        scratch_types=[
            pltpu.VMEM((EPW,), jnp.int32),
            pltpu.VMEM((EPW,), jnp.int32),
            pltpu.VMEM((CH,), jnp.int32),
            pltpu.VMEM((CH,), jnp.int32),
            pltpu.VMEM((CH,), jnp.int32),
            pltpu.VMEM((CH,), jnp.int32),
            pltpu.VMEM((CH,), jnp.int32),
            pltpu.VMEM((NBUF, CH, C), jnp.float32),
            pltpu.VMEM_SHARED((N, C), jnp.float32),
            pltpu.SemaphoreType.DMA((NBUF,)),
            pltpu.SemaphoreType.DMA((NBUF,)),
        ],
        compiler_params=pltpu.CompilerParams(use_tc_tiling_on_sc=False),
    )
    return run(assignments, senders, receivers, zeros)


def _adj_body(a_ref, t_ref, vals_ref, idx_ref):
    t = t_ref[0, :, :C] + t_ref[1, :, :C]
    a = a_ref[...]
    adjs = []
    for g in range(G):
        ag = a[g * NPG:(g + 1) * NPG]
        tg = t[g * NPG:(g + 1) * NPG]
        adjs.append(lax.dot_general(
            ag, tg, (((0,), (0,)), ((), ())),
            preferred_element_type=jnp.float32))
    work = jnp.concatenate(adjs, axis=0)
    jj = lax.broadcasted_iota(jnp.int32, (G * C, C), 1)
    vals_cols, idx_cols = [], []
    for _ in range(K):
        m = jnp.max(work, axis=-1, keepdims=True)
        idx = jnp.argmax(work, axis=-1).astype(jnp.int32)[:, None]
        vals_cols.append(m)
        idx_cols.append(idx)
        work = jnp.where(jj == idx, jnp.finfo(jnp.float32).min, work)
    vals_ref[...] = jnp.concatenate(vals_cols, axis=1)
    idx_ref[...] = jnp.concatenate(idx_cols, axis=1)


def _stage3(assignments, t):
    return pl.pallas_call(
        _adj_body,
        out_shape=[
            jax.ShapeDtypeStruct((G * C, K), jnp.float32),
            jax.ShapeDtypeStruct((G * C, K), jnp.int32),
        ],
        compiler_params=pltpu.CompilerParams(fuse_transposed_lhs_in_matmul=True),
    )(assignments, t)


def kernel(nodes, senders, receivers, n_node, n_edge, W1, b1, W2, b2):
    del n_node, n_edge
    assignments, apack, coarse_nodes, recv_q = _stage1(
        nodes, W1, b1.reshape(1, HID), W2, b2.reshape(1, C), receivers)
    zeros = jnp.zeros((SRW, C), jnp.float32)
    t = _stage2(apack.reshape(N, C), senders, recv_q, zeros)
    top_vals, top_idx = _stage3(assignments, t)
    batch_offset = jnp.arange(G, dtype=jnp.int32)[:, None] * C
    c_senders = (jnp.repeat(jnp.arange(C, dtype=jnp.int32), K)[None, :]
                 + batch_offset).reshape(-1)
    c_receivers = (top_idx.reshape(G, C * K) + batch_offset).reshape(-1)
    c_edge_weights = top_vals.reshape(-1, 1)
    return (coarse_nodes, c_senders, c_receivers, c_edge_weights,
            assignments)
